# baseline (speedup 1.0000x reference)
import functools

import jax
import jax.numpy as jnp
from jax import lax
from jax.experimental import pallas as pl
from jax.experimental.pallas import tpu as pltpu
from jax.experimental.pallas import tpu_sc as plsc


def _proj_body(h_ref, wsrc_ref, wdst_ref, hsrc_out, hdst_out):
    x = h_ref[...]
    dn = (((1,), (1,)), ((), ()))
    hsrc_out[...] = lax.dot_general(x, wsrc_ref[...], dn,
                                    preferred_element_type=jnp.float32)
    hdst_out[...] = lax.dot_general(x, wdst_ref[...], dn,
                                    preferred_element_type=jnp.float32)


def _project(h, W_src, W_dst):
    n, d_in = h.shape
    d_out = W_src.shape[0]
    blk = 1000
    grid = n // blk
    return pl.pallas_call(
        _proj_body,
        grid=(grid,),
        in_specs=[
            pl.BlockSpec((blk, d_in), lambda i: (i, 0)),
            pl.BlockSpec((d_out, d_in), lambda i: (0, 0)),
            pl.BlockSpec((d_out, d_in), lambda i: (0, 0)),
        ],
        out_specs=[
            pl.BlockSpec((blk, d_out), lambda i: (i, 0)),
            pl.BlockSpec((blk, d_out), lambda i: (i, 0)),
        ],
        out_shape=[
            jax.ShapeDtypeStruct((n, d_out), jnp.float32),
            jax.ShapeDtypeStruct((n, d_out), jnp.float32),
        ],
    )(h, W_src, W_dst)


def _make_edge_kernel(e_total, d, epw, chunk, nc, ns):
    nchunk = epw // chunk
    mesh = plsc.VectorSubcoreMesh(core_axis_name="c", subcore_axis_name="s")

    @functools.partial(
        pl.kernel,
        out_type=jax.ShapeDtypeStruct((e_total, d), jnp.float32),
        mesh=mesh,
        scratch_types=[
            pltpu.VMEM((epw,), jnp.int32),
            pltpu.VMEM((epw,), jnp.int32),
            pltpu.VMEM((chunk, d), jnp.float32),
            pltpu.VMEM((chunk, d), jnp.float32),
            pltpu.SemaphoreType.DMA,
            pltpu.SemaphoreType.DMA,
        ],
    )
    def edge_kernel(hsrc_hbm, hdst_hbm, src_hbm, dst_hbm, out_hbm,
                    idx_s, idx_d, rows_a, rows_b, sem_a, sem_b):
        wid = lax.axis_index("s") * nc + lax.axis_index("c")
        base = wid * epw
        pltpu.sync_copy(src_hbm.at[pl.ds(base, epw)], idx_s)
        pltpu.sync_copy(dst_hbm.at[pl.ds(base, epw)], idx_d)

        def chunk_body(j, carry):
            off = j * chunk
            cp_a = pltpu.async_copy(
                hsrc_hbm.at[idx_s.at[pl.ds(off, chunk)]], rows_a, sem_a)
            cp_a.wait()
            cp_b = pltpu.async_copy(
                hdst_hbm.at[idx_d.at[pl.ds(off, chunk)]], rows_a, sem_b,
                add=True)
            cp_b.wait()
            pltpu.sync_copy(rows_a, out_hbm.at[pl.ds(base + off, chunk)])
            return carry

        lax.fori_loop(0, nchunk, chunk_body, 0, unroll=False)

    return edge_kernel


def kernel(h, edge_index, W_src, W_dst):
    n, d_in = h.shape
    e_total = edge_index.shape[1]
    d = W_src.shape[0]

    h_src, h_dst = _project(h, W_src, W_dst)

    info = plsc.get_sparse_core_info()
    nc, ns = info.num_cores, info.num_subcores
    nw = nc * ns
    epw = e_total // nw
    chunk = 80

    src = edge_index[0]
    dst = edge_index[1]

    edge_kernel = _make_edge_kernel(e_total, d, epw, chunk, nc, ns)
    return edge_kernel(h_src, h_dst, src, dst)


# speedup vs baseline: 1.8125x; 1.8125x over previous
import functools

import jax
import jax.numpy as jnp
from jax import lax
from jax.experimental import pallas as pl
from jax.experimental.pallas import tpu as pltpu
from jax.experimental.pallas import tpu_sc as plsc


def _proj_body(h_ref, wsrc_ref, wdst_ref, hsrc_out, hdst_out):
    x = h_ref[...]
    dn = (((1,), (1,)), ((), ()))
    hsrc_out[...] = lax.dot_general(x, wsrc_ref[...], dn,
                                    preferred_element_type=jnp.float32)
    hdst_out[...] = lax.dot_general(x, wdst_ref[...], dn,
                                    preferred_element_type=jnp.float32)


def _project(h, W_src, W_dst):
    n, d_in = h.shape
    d_out = W_src.shape[0]
    blk = 1000
    grid = n // blk
    return pl.pallas_call(
        _proj_body,
        grid=(grid,),
        in_specs=[
            pl.BlockSpec((blk, d_in), lambda i: (i, 0)),
            pl.BlockSpec((d_out, d_in), lambda i: (0, 0)),
            pl.BlockSpec((d_out, d_in), lambda i: (0, 0)),
        ],
        out_specs=[
            pl.BlockSpec((blk, d_out), lambda i: (i, 0)),
            pl.BlockSpec((blk, d_out), lambda i: (i, 0)),
        ],
        out_shape=[
            jax.ShapeDtypeStruct((n, d_out), jnp.float32),
            jax.ShapeDtypeStruct((n, d_out), jnp.float32),
        ],
    )(h, W_src, W_dst)


def _make_edge_kernel(e_total, d, epw, chunk, nc, ns):
    nchunk = epw // chunk
    nsets = 5
    assert nchunk % nsets == 0
    rowbytes = chunk * d * 4
    mesh = plsc.VectorSubcoreMesh(core_axis_name="c", subcore_axis_name="s")

    scratch = (
        [pltpu.VMEM((epw,), jnp.int32)] * 2
        + [pltpu.VMEM((chunk, d), jnp.float32)] * nsets
        + [pltpu.SemaphoreType.DMA] * (3 * nsets)
    )

    @functools.partial(
        pl.kernel,
        out_type=jax.ShapeDtypeStruct((e_total, d), jnp.float32),
        mesh=mesh,
        scratch_types=scratch,
    )
    def edge_kernel(hsrc_hbm, hdst_hbm, src_hbm, dst_hbm, out_hbm, *scr):
        idx_s, idx_d = scr[0], scr[1]
        rows = scr[2:2 + nsets]
        sem_src = scr[2 + nsets:2 + 2 * nsets]
        sem_add = scr[2 + 2 * nsets:2 + 3 * nsets]
        sem_out = scr[2 + 3 * nsets:2 + 4 * nsets]

        wid = lax.axis_index("s") * nc + lax.axis_index("c")
        base = wid * epw
        pltpu.sync_copy(src_hbm.at[pl.ds(base, epw)], idx_s)
        pltpu.sync_copy(dst_hbm.at[pl.ds(base, epw)], idx_d)

        def drain(buf, sem):
            pltpu.make_async_copy(hsrc_hbm.at[pl.ds(0, chunk)], buf, sem).wait()

        def round_body(g, carry):
            for s in range(nsets):
                j = g * nsets + s
                t = (s - 1) % nsets
                u = (s - 2) % nsets

                @pl.when(j < nchunk)
                def _():
                    @pl.when(j >= nsets)
                    def _():
                        drain(rows[s], sem_out[s])
                    pltpu.async_copy(
                        hsrc_hbm.at[idx_s.at[pl.ds(j * chunk, chunk)]],
                        rows[s], sem_src[s])

                @pl.when((j >= 1) & (j <= nchunk))
                def _():
                    drain(rows[t], sem_src[t])
                    pltpu.async_copy(
                        hdst_hbm.at[idx_d.at[pl.ds((j - 1) * chunk, chunk)]],
                        rows[t], sem_add[t], add=True)

                @pl.when((j >= 2) & (j <= nchunk + 1))
                def _():
                    drain(rows[u], sem_add[u])
                    pltpu.async_copy(
                        rows[u],
                        out_hbm.at[pl.ds(base + (j - 2) * chunk, chunk)],
                        sem_out[u])
            return carry

        nrounds = (nchunk + 2 + nsets - 1) // nsets + 1
        lax.fori_loop(0, nrounds, round_body, 0, unroll=False)
        for s in range(nsets):
            drain(rows[s], sem_out[s])

    return edge_kernel


def kernel(h, edge_index, W_src, W_dst):
    n, d_in = h.shape
    e_total = edge_index.shape[1]
    d = W_src.shape[0]

    h_src, h_dst = _project(h, W_src, W_dst)

    info = plsc.get_sparse_core_info()
    nc, ns = info.num_cores, info.num_subcores
    nw = nc * ns
    epw = e_total // nw
    chunk = 80

    src = edge_index[0]
    dst = edge_index[1]

    edge_kernel = _make_edge_kernel(e_total, d, epw, chunk, nc, ns)
    return edge_kernel(h_src, h_dst, src, dst)


# speedup vs baseline: 1.8433x; 1.0170x over previous
import functools

import jax
import jax.numpy as jnp
from jax import lax
from jax.experimental import pallas as pl
from jax.experimental.pallas import tpu as pltpu
from jax.experimental.pallas import tpu_sc as plsc


def _proj_body(h_ref, wsrc_ref, wdst_ref, hsrc_out, hdst_out):
    x = h_ref[...]
    dn = (((1,), (1,)), ((), ()))
    hsrc_out[...] = lax.dot_general(x, wsrc_ref[...], dn,
                                    preferred_element_type=jnp.float32)
    hdst_out[...] = lax.dot_general(x, wdst_ref[...], dn,
                                    preferred_element_type=jnp.float32)


def _project(h, W_src, W_dst):
    n, d_in = h.shape
    d_out = W_src.shape[0]
    blk = 1000
    grid = n // blk
    return pl.pallas_call(
        _proj_body,
        grid=(grid,),
        in_specs=[
            pl.BlockSpec((blk, d_in), lambda i: (i, 0)),
            pl.BlockSpec((d_out, d_in), lambda i: (0, 0)),
            pl.BlockSpec((d_out, d_in), lambda i: (0, 0)),
        ],
        out_specs=[
            pl.BlockSpec((blk, d_out), lambda i: (i, 0)),
            pl.BlockSpec((blk, d_out), lambda i: (i, 0)),
        ],
        out_shape=[
            jax.ShapeDtypeStruct((n, d_out), jnp.float32),
            jax.ShapeDtypeStruct((n, d_out), jnp.float32),
        ],
    )(h, W_src, W_dst)


def _make_edge_kernel(e_total, d, epw, chunk, nc, ns):
    nchunk = epw // chunk
    nsets = 7
    skew_add = 2
    skew_out = 4
    mesh = plsc.VectorSubcoreMesh(core_axis_name="c", subcore_axis_name="s")

    scratch = (
        [pltpu.VMEM((epw,), jnp.int32)] * 2
        + [pltpu.VMEM((chunk, d), jnp.float32)] * nsets
        + [pltpu.SemaphoreType.DMA] * (3 * nsets)
    )

    @functools.partial(
        pl.kernel,
        out_type=jax.ShapeDtypeStruct((e_total, d), jnp.float32),
        mesh=mesh,
        scratch_types=scratch,
    )
    def edge_kernel(hsrc_hbm, hdst_hbm, src_hbm, dst_hbm, out_hbm, *scr):
        idx_s, idx_d = scr[0], scr[1]
        rows = scr[2:2 + nsets]
        sem_src = scr[2 + nsets:2 + 2 * nsets]
        sem_add = scr[2 + 2 * nsets:2 + 3 * nsets]
        sem_out = scr[2 + 3 * nsets:2 + 4 * nsets]

        wid = lax.axis_index("s") * nc + lax.axis_index("c")
        base = wid * epw
        cp_is = pltpu.async_copy(src_hbm.at[pl.ds(base, epw)], idx_s,
                                 sem_src[0])
        cp_id = pltpu.async_copy(dst_hbm.at[pl.ds(base, epw)], idx_d,
                                 sem_src[1])
        cp_is.wait()
        cp_id.wait()

        def drain(buf, sem):
            pltpu.make_async_copy(hsrc_hbm.at[pl.ds(0, chunk)], buf, sem).wait()

        def round_body(g, carry):
            for s in range(nsets):
                j = g * nsets + s
                t = (s - skew_add) % nsets
                u = (s - skew_out) % nsets

                @pl.when(j < nchunk)
                def _():
                    @pl.when(j >= nsets)
                    def _():
                        drain(rows[s], sem_out[s])
                    pltpu.async_copy(
                        hsrc_hbm.at[idx_s.at[pl.ds(j * chunk, chunk)]],
                        rows[s], sem_src[s])

                @pl.when((j >= skew_add) & (j < nchunk + skew_add))
                def _():
                    drain(rows[t], sem_src[t])
                    pltpu.async_copy(
                        hdst_hbm.at[
                            idx_d.at[pl.ds((j - skew_add) * chunk, chunk)]],
                        rows[t], sem_add[t], add=True)

                @pl.when((j >= skew_out) & (j < nchunk + skew_out))
                def _():
                    drain(rows[u], sem_add[u])
                    pltpu.async_copy(
                        rows[u],
                        out_hbm.at[
                            pl.ds(base + (j - skew_out) * chunk, chunk)],
                        sem_out[u])
            return carry

        nrounds = (nchunk + skew_out + nsets - 1) // nsets + 1
        lax.fori_loop(0, nrounds, round_body, 0, unroll=False)
        for s in range(nsets):
            drain(rows[s], sem_out[s])

    return edge_kernel


def kernel(h, edge_index, W_src, W_dst):
    n, d_in = h.shape
    e_total = edge_index.shape[1]
    d = W_src.shape[0]

    h_src, h_dst = _project(h, W_src, W_dst)

    info = plsc.get_sparse_core_info()
    nc, ns = info.num_cores, info.num_subcores
    nw = nc * ns
    epw = e_total // nw
    chunk = 80

    src = edge_index[0]
    dst = edge_index[1]

    edge_kernel = _make_edge_kernel(e_total, d, epw, chunk, nc, ns)
    return edge_kernel(h_src, h_dst, src, dst)


# speedup vs baseline: 2.2047x; 1.1961x over previous
import functools

import jax
import jax.numpy as jnp
from jax import lax
from jax.experimental import pallas as pl
from jax.experimental.pallas import tpu as pltpu
from jax.experimental.pallas import tpu_sc as plsc


def _proj_body(h_ref, wsrc_ref, wdst_ref, hsrc_out, hdst_out):
    x = h_ref[...]
    dn = (((1,), (1,)), ((), ()))
    hsrc_out[...] = lax.dot_general(x, wsrc_ref[...], dn,
                                    preferred_element_type=jnp.float32)
    hdst_out[...] = lax.dot_general(x, wdst_ref[...], dn,
                                    preferred_element_type=jnp.float32)


def _project(h, W_src, W_dst):
    n, d_in = h.shape
    d_out = W_src.shape[0]
    blk = 1000
    grid = n // blk
    return pl.pallas_call(
        _proj_body,
        grid=(grid,),
        in_specs=[
            pl.BlockSpec((blk, d_in), lambda i: (i, 0)),
            pl.BlockSpec((d_out, d_in), lambda i: (0, 0)),
            pl.BlockSpec((d_out, d_in), lambda i: (0, 0)),
        ],
        out_specs=[
            pl.BlockSpec((blk, d_out), lambda i: (i, 0)),
            pl.BlockSpec((blk, d_out), lambda i: (i, 0)),
        ],
        out_shape=[
            jax.ShapeDtypeStruct((n, d_out), jnp.float32),
            jax.ShapeDtypeStruct((n, d_out), jnp.float32),
        ],
    )(h, W_src, W_dst)


def _make_edge_kernel(e_total, d, epw, chunk, nc, ns, n_nodes):
    nchunk = epw // chunk
    nsets = 3
    skew_add = 1
    skew_out = 2
    mesh = plsc.VectorSubcoreMesh(core_axis_name="c", subcore_axis_name="s")

    scratch = (
        [pltpu.VMEM((epw,), jnp.int32)] * 2
        + [pltpu.VMEM((chunk, d), jnp.float32)] * nsets
        + [pltpu.SemaphoreType.DMA] * (3 * nsets)
        + [pltpu.VMEM_SHARED((n_nodes, d), jnp.float32)]
    )

    @functools.partial(
        pl.kernel,
        out_type=jax.ShapeDtypeStruct((e_total, d), jnp.float32),
        mesh=mesh,
        scratch_types=scratch,
    )
    def edge_kernel(hsrc_hbm, hdst_hbm, src_hbm, dst_hbm, out_hbm, *scr):
        idx_s, idx_d = scr[0], scr[1]
        rows = scr[2:2 + nsets]
        sem_src = scr[2 + nsets:2 + 2 * nsets]
        sem_add = scr[2 + 2 * nsets:2 + 3 * nsets]
        sem_out = scr[2 + 3 * nsets:2 + 4 * nsets]
        src_tab = scr[2 + 4 * nsets]

        sid = lax.axis_index("s")
        wid = sid * nc + lax.axis_index("c")
        base = wid * epw
        npr = (n_nodes // ns) // 8 * 8
        tail = n_nodes - npr * ns
        cp_tab = pltpu.async_copy(
            hsrc_hbm.at[pl.ds(sid * npr, npr)],
            src_tab.at[pl.ds(sid * npr, npr)], sem_add[0])
        if tail:
            @pl.when(sid == 0)
            def _():
                pltpu.sync_copy(hsrc_hbm.at[pl.ds(npr * ns, tail)],
                                src_tab.at[pl.ds(npr * ns, tail)])
        cp_is = pltpu.async_copy(src_hbm.at[pl.ds(base, epw)], idx_s,
                                 sem_src[0])
        cp_id = pltpu.async_copy(dst_hbm.at[pl.ds(base, epw)], idx_d,
                                 sem_src[1])
        cp_tab.wait()
        cp_is.wait()
        cp_id.wait()
        plsc.subcore_barrier()

        def drain(buf, sem):
            pltpu.make_async_copy(hsrc_hbm.at[pl.ds(0, chunk)], buf, sem).wait()

        def round_body(g, carry):
            for s in range(nsets):
                j = g * nsets + s
                t = (s - skew_add) % nsets
                u = (s - skew_out) % nsets

                @pl.when(j < nchunk)
                def _():
                    @pl.when(j >= nsets)
                    def _():
                        drain(rows[s], sem_out[s])
                    pltpu.async_copy(
                        src_tab.at[idx_s.at[pl.ds(j * chunk, chunk)]],
                        rows[s], sem_src[s])

                @pl.when((j >= skew_add) & (j < nchunk + skew_add))
                def _():
                    drain(rows[t], sem_src[t])
                    pltpu.async_copy(
                        hdst_hbm.at[
                            idx_d.at[pl.ds((j - skew_add) * chunk, chunk)]],
                        rows[t], sem_add[t], add=True)

                @pl.when((j >= skew_out) & (j < nchunk + skew_out))
                def _():
                    drain(rows[u], sem_add[u])
                    pltpu.async_copy(
                        rows[u],
                        out_hbm.at[
                            pl.ds(base + (j - skew_out) * chunk, chunk)],
                        sem_out[u])
            return carry

        nrounds = (nchunk + skew_out + nsets - 1) // nsets + 1
        lax.fori_loop(0, nrounds, round_body, 0, unroll=False)
        for s in range(nsets):
            drain(rows[s], sem_out[s])

    return edge_kernel


def kernel(h, edge_index, W_src, W_dst):
    n, d_in = h.shape
    e_total = edge_index.shape[1]
    d = W_src.shape[0]

    h_src, h_dst = _project(h, W_src, W_dst)

    info = plsc.get_sparse_core_info()
    nc, ns = info.num_cores, info.num_subcores
    nw = nc * ns
    epw = e_total // nw
    chunk = 80

    src = edge_index[0]
    dst = edge_index[1]

    edge_kernel = _make_edge_kernel(e_total, d, epw, chunk, nc, ns, n)
    return edge_kernel(h_src, h_dst, src, dst)
